# baseline (speedup 1.0000x reference)
import dataclasses

import jax
import jax.numpy as jnp
from jax import lax
from jax.experimental import pallas as pl
from jax.experimental.pallas import tpu as pltpu
from jax.experimental.pallas import tpu_sc as plsc

_SAMPLES_PER_STEP = 4
_LANES = 16
_SPLIT = 32


def _sc_gather_body(ts_hbm, tabs_hbm, ab_hbm, ts_v, tabs_v, ab_v, sem0, sem1):
    cid = lax.axis_index("c")
    sid = lax.axis_index("s")
    B = ts_v.shape[0]

    @pl.when(jnp.logical_and(cid == 0, sid == 0))
    def _():
        cp0 = pltpu.async_copy(ts_hbm, ts_v, sem0)
        cp1 = pltpu.async_copy(tabs_hbm, tabs_v, sem1)
        cp0.wait()
        cp1.wait()
        n_steps = tabs_v.shape[0] // 2
        for k in range(B // _LANES):
            sl = pl.ds(k * _LANES, _LANES)
            idx = ts_v[sl]
            ab_v[sl] = plsc.load_gather(tabs_v, [idx])
            ab_v[pl.ds(B + k * _LANES, _LANES)] = plsc.load_gather(
                tabs_v, [idx + n_steps])
        pltpu.sync_copy(ab_v, ab_hbm)


def _sc_gather(ts, sab, somab):
    B = ts.shape[0]
    n_steps = sab.shape[0]
    tabs = jnp.concatenate([sab, somab])
    mesh = plsc.VectorSubcoreMesh(core_axis_name="c", subcore_axis_name="s")
    cp = pltpu.CompilerParams()
    if "needs_layout_passes" in pltpu.CompilerParams.__dataclass_fields__:
        cp = dataclasses.replace(cp, needs_layout_passes=False)
    gather = pl.kernel(
        _sc_gather_body,
        out_type=jax.ShapeDtypeStruct((2 * B,), jnp.float32),
        mesh=mesh,
        scratch_types=[
            pltpu.VMEM((B,), jnp.int32),
            pltpu.VMEM((2 * n_steps,), jnp.float32),
            pltpu.VMEM((2 * B,), jnp.float32),
            pltpu.SemaphoreType.DMA,
            pltpu.SemaphoreType.DMA,
        ],
        compiler_params=cp,
    )
    return gather(ts, tabs)


def _tc1_body(ts_ref, sab_ref, somab_ref, x_ref, n_ref, o_ref):
    i = pl.program_id(0)
    for j in range(_SAMPLES_PER_STEP):
        t = ts_ref[i * _SAMPLES_PER_STEP + j]
        a = sab_ref[t]
        b = somab_ref[t]
        o_ref[j] = a * x_ref[j] + b * n_ref[j]


def _tc2_body(ab_ref, x_ref, n_ref, prev_ref, o_ref):
    i = pl.program_id(0)
    B = ab_ref.shape[0] // 2
    for j in range(_SAMPLES_PER_STEP):
        s = _SPLIT + i * _SAMPLES_PER_STEP + j
        a = ab_ref[s]
        b = ab_ref[B + s]
        o_ref[j] = a * x_ref[j] + b * n_ref[j]


def kernel(x0, noise, time_steps, sqrt_alpha_bars, sqrt_one_minus_alpha_bars):
    B, C, H, W = x0.shape
    ts = time_steps.astype(jnp.int32)
    blk = (_SAMPLES_PER_STEP, C, H, W)
    ab = _sc_gather(ts, sqrt_alpha_bars, sqrt_one_minus_alpha_bars)
    out1 = pl.pallas_call(
        _tc1_body,
        grid=(_SPLIT // _SAMPLES_PER_STEP,),
        in_specs=[
            pl.BlockSpec(memory_space=pltpu.SMEM),
            pl.BlockSpec(memory_space=pltpu.SMEM),
            pl.BlockSpec(memory_space=pltpu.SMEM),
            pl.BlockSpec(blk, lambda i: (i, 0, 0, 0)),
            pl.BlockSpec(blk, lambda i: (i, 0, 0, 0)),
        ],
        out_specs=pl.BlockSpec(blk, lambda i: (i, 0, 0, 0)),
        out_shape=jax.ShapeDtypeStruct((B, C, H, W), x0.dtype),
    )(ts, sqrt_alpha_bars, sqrt_one_minus_alpha_bars, x0, noise)
    off = _SPLIT // _SAMPLES_PER_STEP
    out = pl.pallas_call(
        _tc2_body,
        grid=((B - _SPLIT) // _SAMPLES_PER_STEP,),
        in_specs=[
            pl.BlockSpec(memory_space=pltpu.SMEM),
            pl.BlockSpec(blk, lambda i: (i + off, 0, 0, 0)),
            pl.BlockSpec(blk, lambda i: (i + off, 0, 0, 0)),
            pl.BlockSpec(memory_space=pl.ANY),
        ],
        out_specs=pl.BlockSpec(blk, lambda i: (i + off, 0, 0, 0)),
        out_shape=jax.ShapeDtypeStruct((B, C, H, W), x0.dtype),
        input_output_aliases={3: 0},
    )(ab, x0, noise, out1)
    return out


# speedup vs baseline: 1.3795x; 1.3795x over previous
import jax
import jax.numpy as jnp
from jax.experimental import pallas as pl
from jax.experimental.pallas import tpu as pltpu


_SAMPLES_PER_STEP = 4


def _ddpm_body(ts_ref, sab_ref, somab_ref, x_ref, n_ref, o_ref):
    i = pl.program_id(0)
    for j in range(_SAMPLES_PER_STEP):
        t = ts_ref[i * _SAMPLES_PER_STEP + j]
        a = sab_ref[t]
        b = somab_ref[t]
        o_ref[j] = a * x_ref[j] + b * n_ref[j]


def kernel(x0, noise, time_steps, sqrt_alpha_bars, sqrt_one_minus_alpha_bars):
    B, C, H, W = x0.shape
    ts = time_steps.astype(jnp.int32)
    out = pl.pallas_call(
        _ddpm_body,
        grid=(B // _SAMPLES_PER_STEP,),
        in_specs=[
            pl.BlockSpec(memory_space=pltpu.SMEM),
            pl.BlockSpec(memory_space=pltpu.SMEM),
            pl.BlockSpec(memory_space=pltpu.SMEM),
            pl.BlockSpec((_SAMPLES_PER_STEP, C, H, W), lambda i: (i, 0, 0, 0)),
            pl.BlockSpec((_SAMPLES_PER_STEP, C, H, W), lambda i: (i, 0, 0, 0)),
        ],
        out_specs=pl.BlockSpec((_SAMPLES_PER_STEP, C, H, W), lambda i: (i, 0, 0, 0)),
        out_shape=jax.ShapeDtypeStruct((B, C, H, W), x0.dtype),
    )(ts, sqrt_alpha_bars, sqrt_one_minus_alpha_bars, x0, noise)
    return out
